# baseline (speedup 1.0000x reference)
import functools

import jax
import jax.numpy as jnp
from jax import lax
from jax.experimental import pallas as pl
from jax.experimental.pallas import tpu as pltpu
from jax.experimental.pallas import tpu_sc as plsc

_NC = 2
_NS = 16
_L = 16
_NW = _NC * _NS


@functools.lru_cache(maxsize=None)
def _make_kernel(B, D):
    assert B % (8 * _NW) == 0 and D % _L == 0
    bpw = B // _NW
    mesh = plsc.VectorSubcoreMesh(core_axis_name="c", subcore_axis_name="s")

    @functools.partial(
        pl.kernel,
        out_type=jax.ShapeDtypeStruct((B,), jnp.float32),
        mesh=mesh,
        scratch_types=[
            pltpu.VMEM((bpw,), jnp.int32),
            pltpu.VMEM((bpw,), jnp.int32),
            pltpu.VMEM((bpw, D), jnp.float32),
            pltpu.VMEM((bpw, D), jnp.float32),
            pltpu.VMEM((bpw,), jnp.float32),
            pltpu.SemaphoreType.DMA,
            pltpu.SemaphoreType.DMA,
        ],
        compiler_params=pltpu.CompilerParams(
            needs_layout_passes=False, use_tc_tiling_on_sc=False),
    )
    def scores_kernel(user_hbm, item_hbm, uid_hbm, iid_hbm, out_hbm,
                      uidx_v, iidx_v, urows_v, vrows_v, out_v, usem, vsem):
        wid = lax.axis_index("s") * _NC + lax.axis_index("c")
        base = wid * bpw
        pltpu.sync_copy(uid_hbm.at[pl.ds(base, bpw)], uidx_v)
        pltpu.sync_copy(iid_hbm.at[pl.ds(base, bpw)], iidx_v)
        cu = pltpu.async_copy(user_hbm.at[uidx_v], urows_v, usem)
        cv = pltpu.async_copy(item_hbm.at[iidx_v], vrows_v, vsem)
        cu.wait()
        cv.wait()

        nchunk = D // _L

        lane = lax.iota(jnp.int32, _L)

        def group(g, carry):
            res = jnp.zeros((_L,), jnp.float32)
            for j in range(_L):
                r = g * _L + j
                acc = urows_v[r, pl.ds(0, _L)] * vrows_v[r, pl.ds(0, _L)]
                for c in range(1, nchunk):
                    acc = acc + (urows_v[r, pl.ds(c * _L, _L)]
                                 * vrows_v[r, pl.ds(c * _L, _L)])
                s = jnp.sum(acc)
                res = jnp.where(lane == j, s, res)
            out_v[pl.ds(g * _L, _L)] = res
            return carry

        lax.fori_loop(0, bpw // _L, group, 0)
        pltpu.sync_copy(out_v, out_hbm.at[pl.ds(base, bpw)])

    return scores_kernel


def kernel(user_table, item_table, user_ids, item_ids):
    B = user_ids.shape[0]
    D = user_table.shape[1]
    f = _make_kernel(B, D)
    return f(user_table, item_table,
             user_ids.astype(jnp.int32), item_ids.astype(jnp.int32))


# speedup vs baseline: 1.5493x; 1.5493x over previous
import functools

import jax
import jax.numpy as jnp
from jax import lax
from jax.experimental import pallas as pl
from jax.experimental.pallas import tpu as pltpu
from jax.experimental.pallas import tpu_sc as plsc

_NC = 2
_NS = 16
_L = 16
_NW = _NC * _NS
_CH = 128


@functools.lru_cache(maxsize=None)
def _make_kernel(B, D):
    assert B % (8 * _NW) == 0 and D % _L == 0
    bpw = B // _NW
    nchunk = bpw // _CH
    mesh = plsc.VectorSubcoreMesh(core_axis_name="c", subcore_axis_name="s")

    @functools.partial(
        pl.kernel,
        out_type=jax.ShapeDtypeStruct((B,), jnp.float32),
        mesh=mesh,
        scratch_types=[
            pltpu.VMEM((bpw,), jnp.int32),
            pltpu.VMEM((bpw,), jnp.int32),
            pltpu.VMEM((_CH, D), jnp.float32),
            pltpu.VMEM((_CH, D), jnp.float32),
            pltpu.VMEM((bpw,), jnp.float32),
            pltpu.SemaphoreType.DMA,
            pltpu.SemaphoreType.DMA,
        ],
        compiler_params=pltpu.CompilerParams(needs_layout_passes=False),
    )
    def scores_kernel(user_hbm, item_hbm, uid_hbm, iid_hbm, out_hbm,
                      uidx_v, iidx_v, ubuf_v, vbuf_v, out_v, usem, vsem):
        wid = lax.axis_index("s") * _NC + lax.axis_index("c")
        base = wid * bpw
        pltpu.sync_copy(uid_hbm.at[pl.ds(base, bpw)], uidx_v)
        pltpu.sync_copy(iid_hbm.at[pl.ds(base, bpw)], iidx_v)

        lane = lax.iota(jnp.int32, _L)

        def chunk_body(g, carry):
            descs = []
            for sub in range(_CH // _L):
                k0 = g * _CH + sub * _L
                uvec = uidx_v[pl.ds(k0, _L)]
                ivec = iidx_v[pl.ds(k0, _L)]
                for j in range(_L):
                    m = sub * _L + j
                    descs.append(pltpu.async_copy(
                        user_hbm.at[uvec[j]], ubuf_v.at[m], usem))
                    descs.append(pltpu.async_copy(
                        item_hbm.at[ivec[j]], vbuf_v.at[m], vsem))
            for d in descs:
                d.wait()

            for sub in range(_CH // _L):
                res = jnp.zeros((_L,), jnp.float32)
                for j in range(_L):
                    m = sub * _L + j
                    acc = ubuf_v[m, pl.ds(0, _L)] * vbuf_v[m, pl.ds(0, _L)]
                    for c in range(1, D // _L):
                        acc = acc + (ubuf_v[m, pl.ds(c * _L, _L)]
                                     * vbuf_v[m, pl.ds(c * _L, _L)])
                    s = jnp.sum(acc)
                    res = jnp.where(lane == j, s, res)
                out_v[pl.ds(g * _CH + sub * _L, _L)] = res
            return carry

        lax.fori_loop(0, nchunk, chunk_body, 0)
        pltpu.sync_copy(out_v, out_hbm.at[pl.ds(base, bpw)])

    return scores_kernel


def kernel(user_table, item_table, user_ids, item_ids):
    B = user_ids.shape[0]
    D = user_table.shape[1]
    f = _make_kernel(B, D)
    return f(user_table, item_table,
             user_ids.astype(jnp.int32), item_ids.astype(jnp.int32))


# speedup vs baseline: 2.1720x; 1.4019x over previous
import functools

import jax
import jax.numpy as jnp
from jax import lax
from jax.experimental import pallas as pl
from jax.experimental.pallas import tpu as pltpu
from jax.experimental.pallas import tpu_sc as plsc

_NC = 2
_NS = 16
_L = 16
_NW = _NC * _NS
_CH = 32
_SUB = 8


@functools.lru_cache(maxsize=None)
def _make_kernel(B, D):
    assert B % (8 * _NW) == 0 and D % _L == 0
    bpw = B // _NW
    nchunk = bpw // _CH
    mesh = plsc.VectorSubcoreMesh(core_axis_name="c", subcore_axis_name="s")

    @functools.partial(
        pl.kernel,
        out_type=jax.ShapeDtypeStruct((B,), jnp.float32),
        mesh=mesh,
        scratch_types=[
            pltpu.VMEM((bpw,), jnp.int32),
            pltpu.VMEM((bpw,), jnp.int32),
            pltpu.VMEM((bpw,), jnp.int32),
            pltpu.VMEM((bpw,), jnp.int32),
            pltpu.VMEM((_CH, _SUB, D), jnp.float32),
            pltpu.VMEM((_CH, _SUB, D), jnp.float32),
            pltpu.VMEM((bpw,), jnp.float32),
            pltpu.SemaphoreType.DMA,
            pltpu.SemaphoreType.DMA,
        ],
        compiler_params=pltpu.CompilerParams(
            needs_layout_passes=False, use_tc_tiling_on_sc=True),
    )
    def scores_kernel(user_hbm, item_hbm, uid_hbm, iid_hbm, out_hbm,
                      uidx_v, iidx_v, utid_v, itid_v, ubuf_v, vbuf_v,
                      out_v, usem, vsem):
        wid = lax.axis_index("s") * _NC + lax.axis_index("c")
        base = wid * bpw
        pltpu.sync_copy(uid_hbm.at[pl.ds(base, bpw)], uidx_v)
        pltpu.sync_copy(iid_hbm.at[pl.ds(base, bpw)], iidx_v)

        def tids(s, carry):
            uvec = uidx_v[pl.ds(s * _L, _L)]
            ivec = iidx_v[pl.ds(s * _L, _L)]
            utid_v[pl.ds(s * _L, _L)] = lax.shift_right_logical(uvec, 3)
            itid_v[pl.ds(s * _L, _L)] = lax.shift_right_logical(ivec, 3)
            return carry

        lax.fori_loop(0, bpw // _L, tids, 0)

        lane = lax.iota(jnp.int32, _L)

        def chunk_body(g, carry):
            descs = []
            for sub in range(_CH // _L):
                k0 = g * _CH + sub * _L
                utvec = utid_v[pl.ds(k0, _L)]
                itvec = itid_v[pl.ds(k0, _L)]
                for j in range(_L):
                    m = sub * _L + j
                    descs.append(pltpu.async_copy(
                        user_hbm.at[utvec[j]], ubuf_v.at[m], usem))
                    descs.append(pltpu.async_copy(
                        item_hbm.at[itvec[j]], vbuf_v.at[m], vsem))
            for d in descs:
                d.wait()

            for sub in range(_CH // _L):
                k0 = g * _CH + sub * _L
                uvec = jnp.bitwise_and(uidx_v[pl.ds(k0, _L)], 7)
                ivec = jnp.bitwise_and(iidx_v[pl.ds(k0, _L)], 7)
                res = jnp.zeros((_L,), jnp.float32)
                for j in range(_L):
                    m = sub * _L + j
                    ru = uvec[j]
                    ri = ivec[j]
                    acc = (ubuf_v[m, ru, pl.ds(0, _L)]
                           * vbuf_v[m, ri, pl.ds(0, _L)])
                    for c in range(1, D // _L):
                        acc = acc + (ubuf_v[m, ru, pl.ds(c * _L, _L)]
                                     * vbuf_v[m, ri, pl.ds(c * _L, _L)])
                    s = jnp.sum(acc)
                    res = jnp.where(lane == j, s, res)
                out_v[pl.ds(k0, _L)] = res
            return carry

        lax.fori_loop(0, nchunk, chunk_body, 0)
        pltpu.sync_copy(out_v, out_hbm.at[pl.ds(base, bpw)])

    return scores_kernel


def kernel(user_table, item_table, user_ids, item_ids):
    B = user_ids.shape[0]
    N, D = user_table.shape
    M = item_table.shape[0]
    u3 = user_table.reshape(N // _SUB, _SUB, D)
    i3 = item_table.reshape(M // _SUB, _SUB, D)
    f = _make_kernel(B, D)
    return f(u3, i3, user_ids.astype(jnp.int32), item_ids.astype(jnp.int32))
